# baseline (speedup 1.0000x reference)
import functools

import jax
import jax.numpy as jnp
from jax import lax
from jax.experimental import pallas as pl
from jax.experimental.pallas import tpu as pltpu
from jax.experimental.pallas import tpu_sc as plsc

NUM_CLASSES = 1000
N_CTX = 16
CTX_DIM = 768
SEQ_LEN = 77
SUF_LEN = SEQ_LEN - 1 - N_CTX
CTX_D = N_CTX * CTX_DIM
SUF_D = SUF_LEN * CTX_DIM
ROW_D = SEQ_LEN * CTX_DIM

_info = plsc.get_sparse_core_info()
_NC = _info.num_cores
_NS = _info.num_subcores
_NW = _NC * _NS


@functools.partial(jax.jit, static_argnames=("batch",))
def _gather_prompts(lab2, ctx2, pre2, suf2, *, batch):
    bpw = batch // _NW
    mesh = plsc.VectorSubcoreMesh(core_axis_name="c", subcore_axis_name="s")

    @functools.partial(
        pl.kernel,
        mesh=mesh,
        out_type=jax.ShapeDtypeStruct((batch, ROW_D), jnp.float32),
        scratch_types=[
            pltpu.VMEM((bpw, 1), jnp.int32),
            pltpu.VMEM((1, CTX_DIM), jnp.float32),
            pltpu.VMEM((1, CTX_D), jnp.float32),
            pltpu.VMEM((1, SUF_D), jnp.float32),
            pltpu.SemaphoreType.DMA,
            pltpu.SemaphoreType.DMA,
        ],
    )
    def body(lab_hbm, ctx_hbm, pre_hbm, suf_hbm, out_hbm,
             idx_v, pre_v, ctx_v, suf_v, sem_c, sem_s):
        wid = lax.axis_index("s") * _NC + lax.axis_index("c")
        base = wid * bpw
        pltpu.sync_copy(lab_hbm.at[pl.ds(base, bpw)], idx_v)
        pltpu.sync_copy(pre_hbm, pre_v)

        def step(j, carry):
            b = base + j
            g_c = pltpu.async_copy(ctx_hbm.at[idx_v.at[j]], ctx_v, sem_c)
            g_s = pltpu.async_copy(suf_hbm.at[idx_v.at[j]], suf_v, sem_s)
            pltpu.sync_copy(pre_v, out_hbm.at[pl.ds(b, 1), pl.ds(0, CTX_DIM)])
            g_c.wait()
            pltpu.sync_copy(ctx_v,
                            out_hbm.at[pl.ds(b, 1), pl.ds(CTX_DIM, CTX_D)])
            g_s.wait()
            pltpu.sync_copy(
                suf_v,
                out_hbm.at[pl.ds(b, 1), pl.ds(CTX_DIM + CTX_D, SUF_D)])
            return carry

        lax.fori_loop(0, bpw, step, 0)

    return body(lab2, ctx2, pre2, suf2)


def kernel(label, cls_ctx, token_prefix, token_suffix):
    batch = label.shape[0]
    lab2 = label.astype(jnp.int32).reshape(batch, 1)
    ctx2 = cls_ctx.reshape(NUM_CLASSES, CTX_D)
    pre2 = token_prefix.reshape(1, CTX_DIM)
    suf2 = token_suffix.reshape(NUM_CLASSES, SUF_D)
    out2 = _gather_prompts(lab2, ctx2, pre2, suf2, batch=batch)
    return out2.reshape(batch, SEQ_LEN, CTX_DIM)


# speedup vs baseline: 1.0020x; 1.0020x over previous
import functools

import jax
import jax.numpy as jnp
from jax import lax
from jax.experimental import pallas as pl
from jax.experimental.pallas import tpu as pltpu
from jax.experimental.pallas import tpu_sc as plsc

NUM_CLASSES = 1000
N_CTX = 16
CTX_DIM = 768
SEQ_LEN = 77
SUF_LEN = SEQ_LEN - 1 - N_CTX
CTX_D = N_CTX * CTX_DIM
SUF_D = SUF_LEN * CTX_DIM
ROW_D = SEQ_LEN * CTX_DIM
CTX_OFF = CTX_DIM
SUF_OFF = CTX_DIM + CTX_D

_info = plsc.get_sparse_core_info()
_NC = _info.num_cores
_NS = _info.num_subcores
_NW = _NC * _NS


@functools.partial(jax.jit, static_argnames=("batch",))
def _gather_prompts(lab2, ctx2, pre2, suf2, *, batch):
    bpw = batch // _NW
    mesh = plsc.VectorSubcoreMesh(core_axis_name="c", subcore_axis_name="s")

    @functools.partial(
        pl.kernel,
        mesh=mesh,
        out_type=jax.ShapeDtypeStruct((batch, ROW_D), jnp.float32),
        scratch_types=[
            pltpu.VMEM((bpw, 1), jnp.int32),
            pltpu.VMEM((1, ROW_D), jnp.float32),
            pltpu.VMEM((1, ROW_D), jnp.float32),
            pltpu.SemaphoreType.DMA,
            pltpu.SemaphoreType.DMA,
            pltpu.SemaphoreType.DMA,
            pltpu.SemaphoreType.DMA,
        ],
    )
    def body(lab_hbm, ctx_hbm, pre_hbm, suf_hbm, out_hbm,
             idx_v, row0, row1, gsem0, gsem1, wsem0, wsem1):
        wid = lax.axis_index("s") * _NC + lax.axis_index("c")
        base = wid * bpw
        rows = (row0, row1)
        gsems = (gsem0, gsem1)
        wsems = (wsem0, wsem1)

        pltpu.sync_copy(lab_hbm.at[pl.ds(base, bpw)], idx_v)
        pltpu.sync_copy(pre_hbm, row0.at[:, pl.ds(0, CTX_DIM)])
        pltpu.sync_copy(pre_hbm, row1.at[:, pl.ds(0, CTX_DIM)])

        def issue_gathers(j, p):
            pltpu.async_copy(ctx_hbm.at[idx_v.at[j]],
                             rows[p].at[:, pl.ds(CTX_OFF, CTX_D)], gsems[p])
            pltpu.async_copy(suf_hbm.at[idx_v.at[j]],
                             rows[p].at[:, pl.ds(SUF_OFF, SUF_D)], gsems[p])

        def drain_gathers(p):
            pltpu.make_async_copy(ctx_hbm.at[pl.ds(0, 1)],
                                  rows[p].at[:, pl.ds(CTX_OFF, CTX_D)],
                                  gsems[p]).wait()
            pltpu.make_async_copy(suf_hbm.at[pl.ds(0, 1)],
                                  rows[p].at[:, pl.ds(SUF_OFF, SUF_D)],
                                  gsems[p]).wait()

        def drain_write(p):
            pltpu.make_async_copy(rows[p], out_hbm.at[pl.ds(0, 1)],
                                  wsems[p]).wait()

        issue_gathers(0, 0)
        npair = bpw // 2

        def pair_step(g, carry):
            j0 = 2 * g

            @pl.when(g >= 1)
            def _():
                drain_write(1)
            issue_gathers(j0 + 1, 1)
            drain_gathers(0)
            pltpu.async_copy(row0, out_hbm.at[pl.ds(base + j0, 1)], wsem0)

            @pl.when(g < npair - 1)
            def _():
                drain_write(0)
                issue_gathers(j0 + 2, 0)
            drain_gathers(1)
            pltpu.async_copy(row1, out_hbm.at[pl.ds(base + j0 + 1, 1)],
                             wsem1)
            return carry

        lax.fori_loop(0, npair, pair_step, 0)
        drain_write(0)
        drain_write(1)

    return body(lab2, ctx2, pre2, suf2)


def kernel(label, cls_ctx, token_prefix, token_suffix):
    batch = label.shape[0]
    lab2 = label.astype(jnp.int32).reshape(batch, 1)
    ctx2 = cls_ctx.reshape(NUM_CLASSES, CTX_D)
    pre2 = token_prefix.reshape(1, CTX_DIM)
    suf2 = token_suffix.reshape(NUM_CLASSES, SUF_D)
    out2 = _gather_prompts(lab2, ctx2, pre2, suf2, batch=batch)
    return out2.reshape(batch, SEQ_LEN, CTX_DIM)


# speedup vs baseline: 3.3560x; 3.3494x over previous
import functools

import jax
import jax.numpy as jnp
from jax import lax
from jax.experimental import pallas as pl
from jax.experimental.pallas import tpu as pltpu
from jax.experimental.pallas import tpu_sc as plsc

NUM_CLASSES = 1000
N_CTX = 16
CTX_DIM = 768
SEQ_LEN = 77
SUF_LEN = SEQ_LEN - 1 - N_CTX
LT = CTX_DIM // 128
U_CTX = NUM_CLASSES * (N_CTX // 8) * LT * 8
U_SUF = SUF_LEN * (NUM_CLASSES // 8) * LT * 8
U_OUT = SEQ_LEN * 128 * LT * 8

try:
    _info = plsc.get_sparse_core_info()
    _NC, _NS = _info.num_cores, _info.num_subcores
except Exception:
    _NC, _NS = 2, 16
_NW = _NC * _NS
UPW = 4 * LT * 8
NBUF = 4


def _unit_indices(label):
    c = label.astype(jnp.int32).reshape(_NW, 1, 4, 1, 8)
    s = jnp.arange(SEQ_LEN, dtype=jnp.int32).reshape(1, SEQ_LEN, 1, 1, 1)
    lv = jnp.arange(LT, dtype=jnp.int32).reshape(1, 1, 1, LT, 1)
    ctx_idx = c * (2 * LT * 8) + ((s - 1) // 8) * (LT * 8) + lv * 8 + (s - 1) % 8
    suf_idx = ((s - 1 - N_CTX) * (NUM_CLASSES // 8) + c // 8) * (LT * 8) \
        + lv * 8 + c % 8
    j = jnp.where(s == 0, lv, jnp.where(s <= N_CTX, ctx_idx, suf_idx))
    j = jnp.broadcast_to(j, (_NW, SEQ_LEN, 4, LT, 8))
    return j.reshape(_NW, SEQ_LEN * 2, 96)


@jax.jit
def _gather_prompts(jidx, a_ctx, a_pre, a_suf):
    mesh = plsc.VectorSubcoreMesh(core_axis_name="c", subcore_axis_name="s")

    @functools.partial(
        pl.kernel,
        mesh=mesh,
        out_type=jax.ShapeDtypeStruct((U_OUT, 128), jnp.float32),
        scratch_types=[
            pltpu.VMEM((SEQ_LEN * 2, 96), jnp.int32),
        ] + [pltpu.VMEM((UPW, 128), jnp.float32)] * NBUF
          + [pltpu.SemaphoreType.DMA] * (2 * NBUF),
    )
    def body(j_hbm, ctx_hbm, pre_hbm, suf_hbm, out_hbm, jv,
             buf0, buf1, buf2, buf3,
             gsem0, gsem1, gsem2, gsem3, wsem0, wsem1, wsem2, wsem3):
        wid = lax.axis_index("s") * _NC + lax.axis_index("c")
        bufs = (buf0, buf1, buf2, buf3)
        gsems = (gsem0, gsem1, gsem2, gsem3)
        wsems = (wsem0, wsem1, wsem2, wsem3)

        pltpu.sync_copy(j_hbm.at[wid], jv)

        def issue_slab(s, k):
            def from_table(tab):
                def _go():
                    for h in range(2):
                        pltpu.async_copy(tab.at[jv.at[2 * s + h]],
                                         bufs[k].at[pl.ds(96 * h, 96)],
                                         gsems[k])
                return _go
            pl.when(s == 0)(from_table(pre_hbm))
            pl.when((s >= 1) & (s <= N_CTX))(from_table(ctx_hbm))
            pl.when(s > N_CTX)(from_table(suf_hbm))

        def drain_gathers(k):
            for h in range(2):
                pltpu.make_async_copy(ctx_hbm.at[pl.ds(0, 96)],
                                      bufs[k].at[pl.ds(96 * h, 96)],
                                      gsems[k]).wait()

        def write_slab(s, k):
            pltpu.async_copy(bufs[k],
                             out_hbm.at[pl.ds(s * (128 * LT * 8) + wid * UPW,
                                              UPW)],
                             wsems[k])

        def drain_write(k):
            pltpu.make_async_copy(bufs[k], out_hbm.at[pl.ds(0, UPW)],
                                  wsems[k]).wait()

        for k in range(NBUF):
            issue_slab(k, k)

        def group(g, carry):
            s0 = NBUF * g
            for k in range(NBUF):
                s = s0 + k
                drain_gathers(k)
                write_slab(s, k)

                @pl.when(s + NBUF < SEQ_LEN)
                def _():
                    drain_write(k)
                    issue_slab(s + NBUF, k)
            return carry

        lax.fori_loop(0, (SEQ_LEN - 1) // NBUF, group, 0)
        drain_gathers(0)
        write_slab(SEQ_LEN - 1, 0)
        for k in range(NBUF):
            drain_write(k)

    return body(jidx, a_ctx, a_pre, a_suf)


def kernel(label, cls_ctx, token_prefix, token_suffix):
    a_ctx = cls_ctx.reshape(NUM_CLASSES, 2, 8, LT, 128).transpose(
        0, 1, 3, 2, 4).reshape(U_CTX, 128)
    a_pre = token_prefix.reshape(LT, 128)
    a_suf = token_suffix.reshape(NUM_CLASSES // 8, 8, SUF_LEN, LT,
                                 128).transpose(2, 0, 3, 1, 4).reshape(
                                     U_SUF, 128)
    o = _gather_prompts(_unit_indices(label), a_ctx, a_pre, a_suf)
    return o.reshape(SEQ_LEN, 128, LT, 8, 128).transpose(
        1, 3, 0, 2, 4).reshape(128 * 8, SEQ_LEN, CTX_DIM)


# speedup vs baseline: 3.8507x; 1.1474x over previous
import functools

import jax
import jax.numpy as jnp
from jax import lax
from jax.experimental import pallas as pl
from jax.experimental.pallas import tpu as pltpu
from jax.experimental.pallas import tpu_sc as plsc

NUM_CLASSES = 1000
N_CTX = 16
CTX_DIM = 768
SEQ_LEN = 77
SUF_LEN = SEQ_LEN - 1 - N_CTX
LT = CTX_DIM // 128
U_CTX = NUM_CLASSES * (N_CTX // 8) * LT * 8
U_SUF = SUF_LEN * (NUM_CLASSES // 8) * LT * 8
U_OUT = SEQ_LEN * 128 * LT * 8
SLAB = 128 * LT * 8

try:
    _info = plsc.get_sparse_core_info()
    _NC, _NS = _info.num_cores, _info.num_subcores
except Exception:
    _NC, _NS = 2, 16
_NW = _NC * _NS
BPW = 1024 // _NW
UPW = (BPW // 8) * LT * 8
NBUF = 4


@jax.jit
def _gather_prompts(lab, a_ctx, a_pre, a_suf):
    mesh = plsc.VectorSubcoreMesh(core_axis_name="c", subcore_axis_name="s")

    @functools.partial(
        pl.kernel,
        mesh=mesh,
        out_type=jax.ShapeDtypeStruct((U_OUT, 128), jnp.float32),
        compiler_params=pltpu.CompilerParams(needs_layout_passes=False),
        scratch_types=[
            pltpu.VMEM((BPW,), jnp.int32),
            pltpu.VMEM((NBUF, 2, 96), jnp.int32),
        ] + [pltpu.VMEM((UPW, 128), jnp.float32)] * NBUF
          + [pltpu.SemaphoreType.DMA] * (2 * NBUF),
    )
    def body(lab_hbm, ctx_hbm, pre_hbm, suf_hbm, out_hbm, labv, jvb,
             buf0, buf1, buf2, buf3,
             gsem0, gsem1, gsem2, gsem3, wsem0, wsem1, wsem2, wsem3):
        wid = lax.axis_index("s") * _NC + lax.axis_index("c")
        bufs = (buf0, buf1, buf2, buf3)
        gsems = (gsem0, gsem1, gsem2, gsem3)
        wsems = (wsem0, wsem1, wsem2, wsem3)

        pltpu.sync_copy(lab_hbm.at[pl.ds(wid * BPW, BPW)], labv)

        iota16 = lax.iota(jnp.int32, 16)
        lane8 = iota16 % 8
        lgrp = iota16 // 8

        def fill_and_issue(s, k):
            s32 = jnp.asarray(s, jnp.int32)

            def each_vreg(fn):
                for h in range(2):
                    for kk in range(LT):
                        bvec = (2 * h + kk // 3) * 8 + lane8
                        lvec = lgrp + (2 * kk) % LT
                        jvb[k, h, pl.ds(16 * kk, 16)] = fn(bvec, lvec)

            def issue(tab):
                for h in range(2):
                    pltpu.async_copy(tab.at[jvb.at[k, h]],
                                     bufs[k].at[pl.ds(96 * h, 96)],
                                     gsems[k])

            @pl.when(s32 == 0)
            def _():
                each_vreg(lambda bvec, lvec: lvec)
                issue(pre_hbm)

            @pl.when((s32 >= 1) & (s32 <= N_CTX))
            def _():
                ctx_base = ((s32 - 1) // 8) * (LT * 8) + (s32 - 1) % 8

                def f(bvec, lvec):
                    c = plsc.load_gather(labv, [bvec])
                    return c * (2 * LT * 8) + lvec * 8 + ctx_base
                each_vreg(f)
                issue(ctx_hbm)

            @pl.when(s32 > N_CTX)
            def _():
                suf_base = (s32 - 1 - N_CTX) * (NUM_CLASSES // 8) * (LT * 8)

                def f(bvec, lvec):
                    c = plsc.load_gather(labv, [bvec])
                    return (c // 8) * (LT * 8) + c % 8 + lvec * 8 + suf_base
                each_vreg(f)
                issue(suf_hbm)

        def drain_gathers(k):
            for h in range(2):
                pltpu.make_async_copy(ctx_hbm.at[pl.ds(0, 96)],
                                      bufs[k].at[pl.ds(96 * h, 96)],
                                      gsems[k]).wait()

        def write_slab(s, k):
            pltpu.async_copy(bufs[k],
                             out_hbm.at[pl.ds(s * SLAB + wid * UPW, UPW)],
                             wsems[k])

        def drain_write(k):
            pltpu.make_async_copy(bufs[k], out_hbm.at[pl.ds(0, UPW)],
                                  wsems[k]).wait()

        for k in range(NBUF):
            fill_and_issue(k, k)

        def group(g, carry):
            s0 = NBUF * g
            for k in range(NBUF):
                s = s0 + k
                drain_gathers(k)
                write_slab(s, k)

                @pl.when(s + NBUF < SEQ_LEN)
                def _():
                    drain_write(k)
                    fill_and_issue(s + NBUF, k)
            return carry

        lax.fori_loop(0, (SEQ_LEN - 1) // NBUF, group, 0)
        drain_gathers(0)
        write_slab(SEQ_LEN - 1, 0)
        for k in range(NBUF):
            drain_write(k)

    return body(lab, a_ctx, a_pre, a_suf)


def kernel(label, cls_ctx, token_prefix, token_suffix):
    a_ctx = cls_ctx.reshape(NUM_CLASSES, 2, 8, LT, 128).transpose(
        0, 1, 3, 2, 4).reshape(U_CTX, 128)
    a_pre = token_prefix.reshape(LT, 128)
    a_suf = token_suffix.reshape(NUM_CLASSES // 8, 8, SUF_LEN, LT,
                                 128).transpose(2, 0, 3, 1, 4).reshape(
                                     U_SUF, 128)
    o = _gather_prompts(label.astype(jnp.int32), a_ctx, a_pre, a_suf)
    return o.reshape(SEQ_LEN, 128, LT, 8, 128).transpose(
        1, 3, 0, 2, 4).reshape(128 * 8, SEQ_LEN, CTX_DIM)
